# baseline (speedup 1.0000x reference)
import functools

import jax
import jax.numpy as jnp
from jax import lax
from jax.experimental import pallas as pl
from jax.experimental.pallas import tpu as pltpu
from jax.experimental.pallas import tpu_sc as plsc

_VOCAB = 1000000
_D = 64
_B = 16384
_H = 50

_NC = 2
_NS = 16
_NW = _NC * _NS
_BAGS_PER_W = _B // _NW
_BAGS_PER_UNIT = 2
_IDX_PER_UNIT = _BAGS_PER_UNIT * _H
_UNITS = _BAGS_PER_W // _BAGS_PER_UNIT
_NBUF = 4
_GROUPS = _UNITS // _NBUF
_NLANE = 16
_DREG = _D // _NLANE


def _accumulate_bag(rows_ref, out_ref, row_base, out_row):
    init = tuple(
        rows_ref[row_base, pl.ds(dd * _NLANE, _NLANE)] for dd in range(_DREG)
    )

    def body(j, accs):
        r = row_base + 1 + j
        return tuple(
            accs[dd] + rows_ref[r, pl.ds(dd * _NLANE, _NLANE)]
            for dd in range(_DREG)
        )

    accs = lax.fori_loop(0, _H - 1, body, init, unroll=7)
    scale = jnp.float32(1.0 / _H)
    for dd in range(_DREG):
        out_ref[out_row, pl.ds(dd * _NLANE, _NLANE)] = jnp.maximum(
            accs[dd] * scale, 0.0
        )


def _bag_body(idx_hbm, w_hbm, out_hbm, idx_v, rows_bufs, out_bufs,
              gather_sems, store_sems):
    wid = lax.axis_index("s") * _NC + lax.axis_index("c")
    base_bag = wid * _BAGS_PER_W

    pltpu.sync_copy(idx_hbm.at[wid], idx_v)

    for b in range(_NBUF):
        pltpu.async_copy(w_hbm.at[idx_v.at[b]], rows_bufs[b], gather_sems[b])

    @pl.loop(0, _GROUPS)
    def _(g):
        for b in range(_NBUF):
            u = g * _NBUF + b
            pltpu.make_async_copy(
                w_hbm.at[idx_v.at[u]], rows_bufs[b], gather_sems[b]
            ).wait()
            @pl.when(g > 0)
            def _():
                pltpu.make_async_copy(
                    out_bufs[b],
                    out_hbm.at[pl.ds(base_bag, _BAGS_PER_UNIT)],
                    store_sems[b],
                ).wait()

            for k in range(_BAGS_PER_UNIT):
                _accumulate_bag(rows_bufs[b], out_bufs[b], k * _H, k)

            pltpu.async_copy(
                out_bufs[b],
                out_hbm.at[
                    pl.ds(base_bag + u * _BAGS_PER_UNIT, _BAGS_PER_UNIT)
                ],
                store_sems[b],
            )

            @pl.when(u + _NBUF < _UNITS)
            def _():
                pltpu.async_copy(
                    w_hbm.at[idx_v.at[u + _NBUF]], rows_bufs[b],
                    gather_sems[b],
                )

    for b in range(_NBUF):
        pltpu.make_async_copy(
            out_bufs[b],
            out_hbm.at[pl.ds(base_bag, _BAGS_PER_UNIT)],
            store_sems[b],
        ).wait()


_NVB_FULL = 7812
_VPAD = 1000064
_FLAT = _VPAD * _D
_TP_NBUF = 3
_TP_TRIPS = 246
_IOTA64 = None


def _tp_body(wt_hbm, flat_hbm, cbufs, obufs, pbuf, in_sems, out_sems):
    wid = lax.axis_index("s") * _NC + lax.axis_index("c")
    iota = lax.iota(jnp.int32, 16)
    ivec64 = iota * 64
    c_perms = [(iota + d) & 15 for d in range(16)]
    st_perms = [ivec64 + ((iota + d) & 15) for d in range(16)]

    def start_in(vb, b):
        pltpu.async_copy(
            wt_hbm.at[:, pl.ds(vb * 128, 128)], cbufs[b], in_sems[b]
        )

    def wait_in(vb, b):
        pltpu.make_async_copy(
            wt_hbm.at[:, pl.ds(vb * 128, 128)], cbufs[b], in_sems[b]
        ).wait()

    def out_desc(vb, b):
        return pltpu.make_async_copy(
            obufs[b], flat_hbm.at[pl.ds(vb * 8192, 8192)], out_sems[b]
        )

    for b in range(_TP_NBUF):
        vb = wid + b * 32

        @pl.when(vb < _NVB_FULL)
        def _():
            start_in(vb, b)

    @pl.loop(0, _TP_TRIPS // _TP_NBUF)
    def _(g):
        for b in range(_TP_NBUF):
            i = g * _TP_NBUF + b
            vb = wid + i * 32

            @pl.when(vb < _NVB_FULL)
            def _():
                wait_in(vb, b)
                @pl.when(i >= _TP_NBUF)
                def _():
                    out_desc(vb, b).wait()

                @plsc.parallel_loop(0, 8, unroll=2)
                def _(lb):
                    l0 = lb * 16
                    l_vec = iota + l0
                    for cblk in range(4):
                        c0 = cblk * 16
                        st_base = l0 * 64 + c0
                        for d in range(16):
                            x = plsc.load_gather(
                                cbufs[b], [c_perms[d] + c0, l_vec]
                            )
                            plsc.store_scatter(
                                obufs[b], [st_perms[d] + st_base], x
                            )
                out_desc(vb, b).start()

                nxt = vb + _TP_NBUF * 32

                @pl.when(nxt < _NVB_FULL)
                def _():
                    start_in(nxt, b)

    for b in range(_TP_NBUF):
        vb = wid + b * 32

        @pl.when(vb < _NVB_FULL)
        def _():
            out_desc(vb, b).wait()

    @pl.when(wid == 0)
    def _():
        for c in range(64):
            pltpu.async_copy(
                wt_hbm.at[c, pl.ds(_NVB_FULL * 128, 64)], pbuf.at[c],
                in_sems[0],
            )
        for c in range(64):
            pltpu.make_async_copy(
                wt_hbm.at[c, pl.ds(_NVB_FULL * 128, 64)], pbuf.at[c],
                in_sems[0],
            ).wait()
        for c in range(64):
            for l0 in range(0, 64, 16):
                x = pbuf[c, pl.ds(l0, 16)]
                plsc.store_scatter(obufs[0], [ivec64 + (l0 * 64 + c)], x)
        pltpu.sync_copy(
            obufs[0].at[pl.ds(0, 4096)],
            flat_hbm.at[pl.ds(_NVB_FULL * 8192, 4096)],
        )


@jax.jit
def _bag_mean_relu(idx, weight):
    mesh = plsc.VectorSubcoreMesh(core_axis_name="c", subcore_axis_name="s")
    tp = pl.kernel(
        _tp_body,
        out_type=jax.ShapeDtypeStruct((_FLAT,), jnp.float32),
        mesh=mesh,
        scratch_types=[
            [pltpu.VMEM((64, 128), jnp.float32) for _ in range(_TP_NBUF)],
            [pltpu.VMEM((8192,), jnp.float32) for _ in range(_TP_NBUF)],
            pltpu.VMEM((64, 64), jnp.float32),
            [pltpu.SemaphoreType.DMA for _ in range(_TP_NBUF)],
            [pltpu.SemaphoreType.DMA for _ in range(_TP_NBUF)],
        ],
        compiler_params=pltpu.CompilerParams(
            use_tc_tiling_on_sc=True,
            needs_layout_passes=False,
            disable_bounds_checks=True,
        ),
    )
    table = tp(weight.T).reshape(_VPAD, _D)
    f = pl.kernel(
        _bag_body,
        out_type=jax.ShapeDtypeStruct((_B, _D), jnp.float32),
        mesh=mesh,
        scratch_types=[
            pltpu.VMEM((_UNITS, _IDX_PER_UNIT), jnp.int32),
            [pltpu.VMEM((_IDX_PER_UNIT, _D), jnp.float32)
             for _ in range(_NBUF)],
            [pltpu.VMEM((_BAGS_PER_UNIT, _D), jnp.float32)
             for _ in range(_NBUF)],
            [pltpu.SemaphoreType.DMA for _ in range(_NBUF)],
            [pltpu.SemaphoreType.DMA for _ in range(_NBUF)],
        ],
        compiler_params=pltpu.CompilerParams(use_tc_tiling_on_sc=False),
    )
    return f(idx, table)


def kernel(indices, weight):
    idx = indices.astype(jnp.int32).reshape(_NW, _UNITS, _IDX_PER_UNIT)
    return _bag_mean_relu(idx, weight)


# speedup vs baseline: 1.7719x; 1.7719x over previous
import functools

import jax
import jax.numpy as jnp
from jax import lax
from jax.experimental import pallas as pl
from jax.experimental.pallas import tpu as pltpu
from jax.experimental.pallas import tpu_sc as plsc

_VOCAB = 1000000
_D = 64
_B = 16384
_H = 50

_NC = 2
_NS = 16
_NW = _NC * _NS
_BAGS_PER_W = _B // _NW
_BAGS_PER_UNIT = 2
_IDX_PER_UNIT = _BAGS_PER_UNIT * _H
_UNITS = _BAGS_PER_W // _BAGS_PER_UNIT
_NBUF = 4
_GROUPS = _UNITS // _NBUF
_NLANE = 16
_DREG = _D // _NLANE


def _accumulate_bag(rows_ref, out_ref, row_base, out_row):
    init = tuple(
        rows_ref[row_base, pl.ds(dd * _NLANE, _NLANE)] for dd in range(_DREG)
    )

    def body(j, accs):
        r = row_base + 1 + j
        return tuple(
            accs[dd] + rows_ref[r, pl.ds(dd * _NLANE, _NLANE)]
            for dd in range(_DREG)
        )

    accs = lax.fori_loop(0, _H - 1, body, init, unroll=7)
    scale = jnp.float32(1.0 / _H)
    for dd in range(_DREG):
        out_ref[out_row, pl.ds(dd * _NLANE, _NLANE)] = jnp.maximum(
            accs[dd] * scale, 0.0
        )


def _bag_body(idx_hbm, w_hbm, out_hbm, idx_v, rows_bufs, out_bufs,
              gather_sems, store_sems):
    wid = lax.axis_index("s") * _NC + lax.axis_index("c")
    base_bag = wid * _BAGS_PER_W

    pltpu.sync_copy(idx_hbm.at[wid], idx_v)

    for b in range(_NBUF):
        pltpu.async_copy(w_hbm.at[idx_v.at[b]], rows_bufs[b], gather_sems[b])

    @pl.loop(0, _GROUPS)
    def _(g):
        for b in range(_NBUF):
            u = g * _NBUF + b
            pltpu.make_async_copy(
                w_hbm.at[idx_v.at[u]], rows_bufs[b], gather_sems[b]
            ).wait()
            @pl.when(g > 0)
            def _():
                pltpu.make_async_copy(
                    out_bufs[b],
                    out_hbm.at[pl.ds(base_bag, _BAGS_PER_UNIT)],
                    store_sems[b],
                ).wait()

            for k in range(_BAGS_PER_UNIT):
                _accumulate_bag(rows_bufs[b], out_bufs[b], k * _H, k)

            pltpu.async_copy(
                out_bufs[b],
                out_hbm.at[
                    pl.ds(base_bag + u * _BAGS_PER_UNIT, _BAGS_PER_UNIT)
                ],
                store_sems[b],
            )

            @pl.when(u + _NBUF < _UNITS)
            def _():
                pltpu.async_copy(
                    w_hbm.at[idx_v.at[u + _NBUF]], rows_bufs[b],
                    gather_sems[b],
                )

    for b in range(_NBUF):
        pltpu.make_async_copy(
            out_bufs[b],
            out_hbm.at[pl.ds(base_bag, _BAGS_PER_UNIT)],
            store_sems[b],
        ).wait()


_NVB_FULL = 7812
_VPAD = 1000064
_FLAT = _VPAD * _D
_TP_NBUF = 3
_TP_TRIPS = 246
_IOTA64 = None


def _tp_body(wt_hbm, flat_hbm, cbufs, obufs, pbuf, in_sems, out_sems):
    wid = lax.axis_index("s") * _NC + lax.axis_index("c")
    iota = lax.iota(jnp.int32, 16)
    ivec64 = iota * 64
    c_perms = [(iota + d) & 15 for d in range(16)]
    st_perms = [ivec64 + ((iota + d) & 15) for d in range(16)]

    def start_in(vb, b):
        pltpu.async_copy(
            wt_hbm.at[:, pl.ds(vb * 128, 128)], cbufs[b], in_sems[b]
        )

    def wait_in(vb, b):
        pltpu.make_async_copy(
            wt_hbm.at[:, pl.ds(vb * 128, 128)], cbufs[b], in_sems[b]
        ).wait()

    def out_desc(vb, b):
        return pltpu.make_async_copy(
            obufs[b], flat_hbm.at[pl.ds(vb * 8192, 8192)], out_sems[b]
        )

    for b in range(_TP_NBUF):
        vb = wid + b * 32

        @pl.when(vb < _NVB_FULL)
        def _():
            start_in(vb, b)

    @pl.loop(0, _TP_TRIPS // _TP_NBUF)
    def _(g):
        for b in range(_TP_NBUF):
            i = g * _TP_NBUF + b
            vb = wid + i * 32

            @pl.when(vb < _NVB_FULL)
            def _():
                wait_in(vb, b)
                @pl.when(i >= _TP_NBUF)
                def _():
                    out_desc(vb, b).wait()

                @pl.loop(0, 8, unroll=2)
                def _(lb):
                    l0 = lb * 16
                    l_vec = iota + l0
                    for cblk in range(4):
                        c0 = cblk * 16
                        st_base = l0 * 64 + c0
                        for d in range(16):
                            x = plsc.load_gather(
                                cbufs[b], [c_perms[d] + c0, l_vec]
                            )
                            plsc.store_scatter(
                                obufs[b], [st_perms[d] + st_base], x
                            )
                out_desc(vb, b).start()

                nxt = vb + _TP_NBUF * 32

                @pl.when(nxt < _NVB_FULL)
                def _():
                    start_in(nxt, b)

    for b in range(_TP_NBUF):
        vb = wid + b * 32

        @pl.when(vb < _NVB_FULL)
        def _():
            out_desc(vb, b).wait()

    @pl.when(wid == 0)
    def _():
        for c in range(64):
            pltpu.async_copy(
                wt_hbm.at[c, pl.ds(_NVB_FULL * 128, 64)], pbuf.at[c],
                in_sems[0],
            )
        for c in range(64):
            pltpu.make_async_copy(
                wt_hbm.at[c, pl.ds(_NVB_FULL * 128, 64)], pbuf.at[c],
                in_sems[0],
            ).wait()
        for c in range(64):
            for l0 in range(0, 64, 16):
                x = pbuf[c, pl.ds(l0, 16)]
                plsc.store_scatter(obufs[0], [ivec64 + (l0 * 64 + c)], x)
        pltpu.sync_copy(
            obufs[0].at[pl.ds(0, 4096)],
            flat_hbm.at[pl.ds(_NVB_FULL * 8192, 4096)],
        )


@jax.jit
def _bag_mean_relu(idx, weight):
    mesh = plsc.VectorSubcoreMesh(core_axis_name="c", subcore_axis_name="s")
    tp = pl.kernel(
        _tp_body,
        out_type=jax.ShapeDtypeStruct((_FLAT,), jnp.float32),
        mesh=mesh,
        scratch_types=[
            [pltpu.VMEM((64, 128), jnp.float32) for _ in range(_TP_NBUF)],
            [pltpu.VMEM((8192,), jnp.float32) for _ in range(_TP_NBUF)],
            pltpu.VMEM((64, 64), jnp.float32),
            [pltpu.SemaphoreType.DMA for _ in range(_TP_NBUF)],
            [pltpu.SemaphoreType.DMA for _ in range(_TP_NBUF)],
        ],
        compiler_params=pltpu.CompilerParams(
            use_tc_tiling_on_sc=True,
            needs_layout_passes=False,
            disable_bounds_checks=True,
        ),
    )
    table = tp(weight.T).reshape(_VPAD, _D)
    f = pl.kernel(
        _bag_body,
        out_type=jax.ShapeDtypeStruct((_B, _D), jnp.float32),
        mesh=mesh,
        scratch_types=[
            pltpu.VMEM((_UNITS, _IDX_PER_UNIT), jnp.int32),
            [pltpu.VMEM((_IDX_PER_UNIT, _D), jnp.float32)
             for _ in range(_NBUF)],
            [pltpu.VMEM((_BAGS_PER_UNIT, _D), jnp.float32)
             for _ in range(_NBUF)],
            [pltpu.SemaphoreType.DMA for _ in range(_NBUF)],
            [pltpu.SemaphoreType.DMA for _ in range(_NBUF)],
        ],
        compiler_params=pltpu.CompilerParams(use_tc_tiling_on_sc=False),
    )
    return f(idx, table)


def kernel(indices, weight):
    idx = indices.astype(jnp.int32).reshape(_NW, _UNITS, _IDX_PER_UNIT)
    return _bag_mean_relu(idx, weight)
